# baseline (speedup 1.0000x reference)
import jax
import jax.numpy as jnp
from jax.experimental import pallas as pl
from jax.experimental.pallas import tpu as pltpu


def _copy_body(f_in, m_in, f_out, m_out, mask_out):
    f_out[...] = f_in[...]
    m_out[...] = m_in[...]
    mask_out[...] = jnp.ones(mask_out.shape, dtype=jnp.bool_)


def kernel(features, means, xy_coords, A):
    B, V, G, C = features.shape
    del xy_coords, A
    BV = B * V
    f2 = features.reshape(BV * G, C)
    m3 = means.reshape(BV, 1, G * 3)

    f_out, m_out, mask = pl.pallas_call(
        _copy_body,
        grid=(BV,),
        in_specs=[
            pl.BlockSpec((G, C), lambda i: (i, 0)),
            pl.BlockSpec((1, 1, G * 3), lambda i: (i, 0, 0)),
        ],
        out_specs=[
            pl.BlockSpec((G, C), lambda i: (i, 0)),
            pl.BlockSpec((1, 1, G * 3), lambda i: (i, 0, 0)),
            pl.BlockSpec((1, 1, G), lambda i: (i, 0, 0)),
        ],
        out_shape=[
            jax.ShapeDtypeStruct((BV * G, C), features.dtype),
            jax.ShapeDtypeStruct((BV, 1, G * 3), means.dtype),
            jax.ShapeDtypeStruct((BV, 1, G), jnp.bool_),
        ],
        compiler_params=pltpu.CompilerParams(
            dimension_semantics=("parallel",),
        ),
    )(f2, m3)

    return (
        f_out.reshape(B, V * G, C),
        m_out.reshape(B, V * G, 3),
        mask.reshape(B, V, G),
    )
